# Initial kernel scaffold; baseline (speedup 1.0000x reference)
import jax
import jax.numpy as jnp
from jax.experimental import pallas as pl


def kernel(query, queue_anchor, queue_label):
    raise NotImplementedError("write your pallas kernel here")


# speedup vs baseline: 2.0841x; 2.0841x over previous
import functools

import jax
import jax.numpy as jnp
from jax import lax
from jax.experimental import pallas as pl
from jax.experimental.pallas import tpu as pltpu

K = 1024
DIM = 2048
KNN = 8
Q = 32
KT = 256
NSTEPS = K // KT


def _tc_body(query_ref, anchor_ref, label_ref, out_ref, dist_ref, logq_ref):
    i = pl.program_id(0)

    @pl.when(i == 0)
    def _():
        logq_ref[...] = jnp.log(query_ref[...])

    a = anchor_ref[...]
    log_a = jnp.log(a)
    self_term = jnp.sum(a * log_a, axis=1, keepdims=True)
    cross = lax.dot_general(
        a, logq_ref[...], (((1,), (1,)), ((), ())),
        preferred_element_type=jnp.float32)
    dist_ref[pl.ds(i * KT, KT), :] = (self_term - cross) / DIM

    @pl.when(i == NSTEPS - 1)
    def _():
        d = dist_ref[...]
        iota = lax.broadcasted_iota(jnp.int32, (K, Q), 0)
        labels = label_ref[...]
        s = jnp.zeros((1, Q), jnp.float32)
        for _ in range(KNN):
            m = jnp.min(d, axis=0, keepdims=True)
            idx = jnp.min(jnp.where(d == m, iota, K), axis=0, keepdims=True)
            sel = iota == idx
            s = s + jnp.sum(jnp.where(sel, labels, 0.0), axis=0, keepdims=True)
            d = jnp.where(sel, jnp.inf, d)
        out_ref[...] = (s >= KNN / 2 + 0.5).astype(jnp.int32)


@jax.jit
def kernel(query, queue_anchor, queue_label):
    labels_f = queue_label.astype(jnp.float32).reshape(K, 1)
    out = pl.pallas_call(
        _tc_body,
        grid=(NSTEPS,),
        in_specs=[
            pl.BlockSpec((Q, DIM), lambda i: (0, 0)),
            pl.BlockSpec((KT, DIM), lambda i: (i, 0)),
            pl.BlockSpec((K, 1), lambda i: (0, 0)),
        ],
        out_specs=pl.BlockSpec((1, Q), lambda i: (0, 0)),
        out_shape=jax.ShapeDtypeStruct((1, Q), jnp.int32),
        scratch_shapes=[
            pltpu.VMEM((K, Q), jnp.float32),
            pltpu.VMEM((Q, DIM), jnp.float32),
        ],
    )(query, queue_anchor, labels_f)
    return out.reshape(Q)
